# baseline (speedup 1.0000x reference)
import functools

import jax
import jax.numpy as jnp
from jax import lax
from jax.experimental import pallas as pl
from jax.experimental.pallas import tpu as pltpu
from jax.experimental.pallas import tpu_sc as plsc

MAX_PATH = 512
D = 32
DP = 128
B0, N = 8, 512
NC, NS, L = 2, 16, 16
NW = NC * NS
W_PER_B = NW // B0
ROWS_W = N // W_PER_B
CJ = 128
HALVES = N // CJ
NCHUNK = ROWS_W * HALVES
CXR = CJ * D // DP
XROWS = B0 * N * N * D // DP

_mesh = plsc.VectorSubcoreMesh(core_axis_name="c", subcore_axis_name="s")


@functools.partial(
    pl.kernel,
    out_type=jax.ShapeDtypeStruct((XROWS, DP), jnp.float32),
    mesh=_mesh,
    scratch_types=[
        pltpu.VMEM((CJ,), jnp.int32),
        pltpu.VMEM((CJ,), jnp.int32),
        pltpu.VMEM((CJ, D), jnp.float32),
        pltpu.VMEM((CJ, D), jnp.float32),
        pltpu.VMEM((CXR, DP), jnp.float32),
        pltpu.VMEM((CXR, DP), jnp.float32),
        pltpu.SemaphoreType.DMA,
        pltpu.SemaphoreType.DMA,
        pltpu.SemaphoreType.DMA,
        pltpu.SemaphoreType.DMA,
    ],
    compiler_params=pltpu.CompilerParams(use_tc_tiling_on_sc=False),
)
def _lookup(idx_hbm, table_hbm, out_hbm,
            idx0, idx1, rows0, rows1, cmp0, cmp1,
            gs0, gs1, ws0, ws1):
    idxb = (idx0, idx1)
    rowsb = (rows0, rows1)
    cmpb = (cmp0, cmp1)
    gs = (gs0, gs1)
    ws = (ws0, ws1)

    wid = lax.axis_index("s") * NC + lax.axis_index("c")
    bi = wid // W_PER_B
    r0 = (wid % W_PER_B) * ROWS_W
    toff = wid * MAX_PATH
    xr0 = (bi * N + r0) * (N * D // DP)

    def stage(c, b):
        ri = r0 + c // HALVES
        j0 = (c % HALVES) * CJ
        pltpu.sync_copy(idx_hbm.at[bi, ri, pl.ds(j0, CJ)], idxb[b])
        for k in range(CJ // L):
            v = idxb[b][pl.ds(k * L, L)]
            idxb[b][pl.ds(k * L, L)] = (
                jnp.minimum(jnp.maximum(v, 0), MAX_PATH - 1) + toff)
        pltpu.async_copy(table_hbm.at[idxb[b]], rowsb[b], gs[b])

    def drain_gather(b):
        pltpu.make_async_copy(table_hbm.at[idxb[b]], rowsb[b], gs[b]).wait()

    def compact(b):
        def rows4(r4, _):
            for rr in range(4):
                r = r4 * 4 + rr
                for k in range(D // L):
                    cmpb[b][r4, pl.ds(rr * D + k * L, L)] = (
                        rowsb[b][r, pl.ds(k * L, L)])
            return 0

        lax.fori_loop(0, CXR, rows4, 0)

    def fire_writeback(c, b):
        pltpu.async_copy(cmpb[b], out_hbm.at[pl.ds(xr0 + c * CXR, CXR)],
                         ws[b])

    def wait_writeback(c, b):
        pltpu.make_async_copy(
            cmpb[b], out_hbm.at[pl.ds(xr0 + c * CXR, CXR)], ws[b]).wait()

    stage(0, 0)

    def pair_body(g2, _):
        g = g2 * 2
        for b in range(2):
            c = g + b
            nb = 1 - b
            drain_gather(b)
            compact(b)
            fire_writeback(c, b)

            @pl.when(c + 1 < NCHUNK)
            def _():
                @pl.when(c >= 1)
                def _():
                    wait_writeback(c - 1, nb)

                stage(c + 1, nb)

        return 0

    lax.fori_loop(0, NCHUNK // 2, pair_body, 0)

    wait_writeback(NCHUNK - 2, 0)
    wait_writeback(NCHUNK - 1, 1)


def _retile_body(x_ref, o_ref):
    xb = x_ref[...]
    parts = [xb[:, k * D:(k + 1) * D] for k in range(DP // D)]
    o_ref[...] = jnp.stack(parts, axis=1).reshape(o_ref.shape)


RG = 8
_retile = pl.pallas_call(
    _retile_body,
    out_shape=jax.ShapeDtypeStruct((B0, N, N, D), jnp.float32),
    grid=(B0, N // RG),
    in_specs=[pl.BlockSpec((RG * N * D // DP, DP),
                           lambda b, i: (b * (N // RG) + i, 0))],
    out_specs=pl.BlockSpec((1, RG, N, D), lambda b, i: (b, i, 0, 0)),
)


def kernel(spatial_matrix, spatial_embedding):
    table_rep = jnp.tile(spatial_embedding, (NW, 1))
    x = _lookup(spatial_matrix, table_rep)
    return _retile(x)


# speedup vs baseline: 2.6278x; 2.6278x over previous
import functools

import jax
import jax.numpy as jnp
from jax import lax
from jax.experimental import pallas as pl
from jax.experimental.pallas import tpu as pltpu
from jax.experimental.pallas import tpu_sc as plsc

MAX_PATH = 512
D = 32
B0, N = 8, 512
NC, NS, L = 2, 16, 16
NW = NC * NS
W_PER_B = NW // B0
ROWS_W = N // W_PER_B
CR = 2
CHUNK = CR * N
ROWS = CHUNK // 128
NCHUNK = ROWS_W // CR

_mesh = plsc.VectorSubcoreMesh(core_axis_name="c", subcore_axis_name="s")


@functools.partial(
    pl.kernel,
    out_type=jax.ShapeDtypeStruct((B0, N, N, D), jnp.float32),
    mesh=_mesh,
    scratch_types=[
        pltpu.VMEM((CR, N), jnp.int32),
        pltpu.VMEM((CR, N), jnp.int32),
        pltpu.VMEM((CR, N, D), jnp.float32),
        pltpu.VMEM((CR, N, D), jnp.float32),
        pltpu.SemaphoreType.DMA,
        pltpu.SemaphoreType.DMA,
        pltpu.SemaphoreType.DMA,
        pltpu.SemaphoreType.DMA,
    ],
    compiler_params=pltpu.CompilerParams(use_tc_tiling_on_sc=False),
)
def _lookup(idx_hbm, table_hbm, out_hbm,
            idx0, idx1, rows0, rows1, gs0, gs1, ws0, ws1):
    idxb = (idx0, idx1)
    rowsb = (rows0, rows1)
    gs = (gs0, gs1)
    ws = (ws0, ws1)

    wid = lax.axis_index("s") * NC + lax.axis_index("c")
    bi = wid // W_PER_B
    r0 = (wid % W_PER_B) * ROWS_W
    toff = wid * MAX_PATH

    def stage(c, b):
        pltpu.sync_copy(idx_hbm.at[bi, pl.ds(r0 + c * CR, CR)], idxb[b])
        for r in range(CR):
            for k in range(N // L):
                v = idxb[b][r, pl.ds(k * L, L)]
                idxb[b][r, pl.ds(k * L, L)] = (
                    jnp.minimum(jnp.maximum(v, 0), MAX_PATH - 1) + toff)
        for j in range(ROWS):
            r, k = divmod(j, N // 128)
            pltpu.async_copy(
                table_hbm.at[idxb[b].at[r, pl.ds(k * 128, 128)]],
                rowsb[b].at[r, pl.ds(k * 128, 128)],
                gs[b],
            )

    def drain_gather(b):
        for j in range(ROWS):
            r, k = divmod(j, N // 128)
            pltpu.make_async_copy(
                table_hbm.at[idxb[b].at[r, pl.ds(k * 128, 128)]],
                rowsb[b].at[r, pl.ds(k * 128, 128)],
                gs[b],
            ).wait()

    def fire_writeback(c, b):
        pltpu.async_copy(rowsb[b], out_hbm.at[bi, pl.ds(r0 + c * CR, CR)],
                         ws[b])

    def wait_writeback(c, b):
        pltpu.make_async_copy(
            rowsb[b], out_hbm.at[bi, pl.ds(r0 + c * CR, CR)], ws[b]).wait()

    stage(0, 0)

    def pair_body(g2, _):
        g = g2 * 2
        for b in range(2):
            c = g + b
            nb = 1 - b
            drain_gather(b)
            fire_writeback(c, b)

            @pl.when(c + 1 < NCHUNK)
            def _():
                @pl.when(c >= 1)
                def _():
                    wait_writeback(c - 1, nb)

                stage(c + 1, nb)

        return 0

    lax.fori_loop(0, NCHUNK // 2, pair_body, 0)

    wait_writeback(NCHUNK - 2, 0)
    wait_writeback(NCHUNK - 1, 1)


def kernel(spatial_matrix, spatial_embedding):
    table_rep = jnp.tile(spatial_embedding, (NW, 1))
    return _lookup(spatial_matrix, table_rep)
